# Initial kernel scaffold; baseline (speedup 1.0000x reference)
import jax
import jax.numpy as jnp
from jax import lax
from jax.experimental import pallas as pl
from jax.experimental.pallas import tpu as pltpu
from jax.experimental.pallas import tpu_sc as plsc

NUM_FIXED = 2
D = 64
BATCH = 16384
SEQ = 26
B = BATCH * SEQ
NC, NS, L = 2, 16, 16
NW = NC * NS
B_PER_W = B // NW
SUP = 1024
N_SUP = B_PER_W // SUP
G_PER_SUP = SUP // 128


def _body(idx_hbm, freeze_hbm, train_hbm, out_hbm,
          idx_v, idxc_v, rows_v, freeze_v, gsem):
    wid = lax.axis_index("s") * NC + lax.axis_index("c")
    pltpu.sync_copy(freeze_hbm, freeze_v)

    def superchunk(s, carry):
        rowblk = wid * (N_SUP * G_PER_SUP) + s * G_PER_SUP
        base = rowblk * 128
        pltpu.sync_copy(idx_hbm.at[pl.ds(rowblk, G_PER_SUP)], idx_v)

        for r in range(G_PER_SUP):
            for k in range(128 // L):
                iv = idx_v[r, pl.ds(k * L, L)]
                idxc_v[r, pl.ds(k * L, L)] = jnp.maximum(iv - NUM_FIXED, 0)

        cps = [
            pltpu.async_copy(
                train_hbm.at[idxc_v.at[j]],
                rows_v.at[pl.ds(j * 128, 128)],
                gsem,
            )
            for j in range(G_PER_SUP)
        ]
        for cp in cps:
            cp.wait()

        def fix(g, c):
            r = g // (128 // L)
            k = (g % (128 // L)) * L
            iv = idx_v[r, pl.ds(k, L)]
            m = iv < NUM_FIXED

            @pl.when(jnp.any(m))
            def _():
                p = g * L + lax.iota(jnp.int32, L)
                ivc = jnp.minimum(iv, NUM_FIXED - 1)
                for col in range(D):
                    cvec = jnp.full((L,), col, jnp.int32)
                    v = plsc.load_gather(freeze_v, [ivc, cvec])
                    plsc.store_scatter(rows_v, [p, cvec], v, mask=m)

            return c

        lax.fori_loop(0, SUP // L, fix, 0)

        pltpu.sync_copy(rows_v, out_hbm.at[pl.ds(base, SUP)])
        return carry

    lax.fori_loop(0, N_SUP, superchunk, 0)


@jax.jit
def _gather(idx2d, weights_freeze, weights_train):
    mesh = plsc.VectorSubcoreMesh(core_axis_name="c", subcore_axis_name="s")
    f = pl.kernel(
        _body,
        out_type=jax.ShapeDtypeStruct((B, D), jnp.float32),
        mesh=mesh,
        scratch_types=[
            pltpu.VMEM((G_PER_SUP, 128), jnp.int32),
            pltpu.VMEM((G_PER_SUP, 128), jnp.int32),
            pltpu.VMEM((SUP, D), jnp.float32),
            pltpu.VMEM((NUM_FIXED, D), jnp.float32),
            pltpu.SemaphoreType.DMA,
        ],
    )
    return f(idx2d, weights_freeze, weights_train)


def kernel(idx, weights_freeze, weights_train):
    idx2d = idx.astype(jnp.int32).reshape(B // 128, 128)
    out = _gather(idx2d, weights_freeze.astype(jnp.float32),
                  weights_train.astype(jnp.float32))
    return out.reshape(BATCH, SEQ, D)


# speedup vs baseline: 1.2075x; 1.2075x over previous
import jax
import jax.numpy as jnp
from jax import lax
from jax.experimental import pallas as pl
from jax.experimental.pallas import tpu as pltpu
from jax.experimental.pallas import tpu_sc as plsc

NUM_FIXED = 2
D = 64
BATCH = 16384
SEQ = 26
B = BATCH * SEQ
NC, NS, L = 2, 16, 16
NW = NC * NS
B_PER_W = B // NW
SUP = 1024
N_SUP = B_PER_W // SUP
G_PER_SUP = SUP // 128


def _body(idx_hbm, freeze_hbm, train_hbm, out_hbm,
          idx_v, idxc_v, rows_v, freeze_v, gsem):
    wid = lax.axis_index("s") * NC + lax.axis_index("c")
    pltpu.sync_copy(freeze_hbm, freeze_v)

    def superchunk(s, carry):
        rowblk = wid * (N_SUP * G_PER_SUP) + s * G_PER_SUP
        base = rowblk * 128
        pltpu.sync_copy(idx_hbm.at[pl.ds(rowblk, G_PER_SUP)], idx_v)

        for r in range(G_PER_SUP):
            for k in range(128 // L):
                iv = idx_v[r, pl.ds(k * L, L)]
                idxc_v[r, pl.ds(k * L, L)] = jnp.maximum(iv - NUM_FIXED, 0)

        cps = [
            pltpu.async_copy(
                train_hbm.at[idxc_v.at[j]],
                rows_v.at[pl.ds(j * 128, 128)],
                gsem,
            )
            for j in range(G_PER_SUP)
        ]
        for cp in cps:
            cp.wait()

        def fix(g, c):
            r = g // (128 // L)
            k = (g % (128 // L)) * L
            iv = idx_v[r, pl.ds(k, L)]
            m = iv < NUM_FIXED

            @pl.when(plsc.all_reduce_population_count(m)[0] > 0)
            def _():
                p = g * L + lax.iota(jnp.int32, L)
                ivc = jnp.minimum(iv, NUM_FIXED - 1)
                for col in range(D):
                    cvec = jnp.full((L,), col, jnp.int32)
                    v = plsc.load_gather(freeze_v, [ivc, cvec])
                    plsc.store_scatter(rows_v, [p, cvec], v, mask=m)

            return c

        lax.fori_loop(0, SUP // L, fix, 0)

        pltpu.sync_copy(rows_v, out_hbm.at[pl.ds(base, SUP)])
        return carry

    lax.fori_loop(0, N_SUP, superchunk, 0)


@jax.jit
def _gather(idx2d, weights_freeze, weights_train):
    mesh = plsc.VectorSubcoreMesh(core_axis_name="c", subcore_axis_name="s")
    f = pl.kernel(
        _body,
        out_type=jax.ShapeDtypeStruct((B, D), jnp.float32),
        mesh=mesh,
        scratch_types=[
            pltpu.VMEM((G_PER_SUP, 128), jnp.int32),
            pltpu.VMEM((G_PER_SUP, 128), jnp.int32),
            pltpu.VMEM((SUP, D), jnp.float32),
            pltpu.VMEM((NUM_FIXED, D), jnp.float32),
            pltpu.SemaphoreType.DMA,
        ],
        compiler_params=pltpu.CompilerParams(
            needs_layout_passes=False, use_tc_tiling_on_sc=False),
    )
    return f(idx2d, weights_freeze, weights_train)


def kernel(idx, weights_freeze, weights_train):
    idx2d = idx.astype(jnp.int32).reshape(B // 128, 128)
    out = _gather(idx2d, weights_freeze.astype(jnp.float32),
                  weights_train.astype(jnp.float32))
    return out.reshape(BATCH, SEQ, D)
